# Initial kernel scaffold; baseline (speedup 1.0000x reference)
import jax
import jax.numpy as jnp
from jax.experimental import pallas as pl


def kernel(z, neighbor_indices, adjacency, basis_weights, channel_coeffs):
    raise NotImplementedError("write your pallas kernel here")


# speedup vs baseline: 2.5784x; 2.5784x over previous
import functools

import jax
import jax.numpy as jnp
from jax import lax
from jax.experimental import pallas as pl
from jax.experimental.pallas import tpu as pltpu
from jax.experimental.pallas import tpu_sc as plsc

NC = 2
NS = 16
NW = NC * NS
LANES = 16
NB = 8


def _tanh16(x):
    ax = jnp.abs(x)
    e = jnp.exp(-2.0 * ax)
    t = (1.0 - e) / (1.0 + e)
    return jnp.where(x < 0.0, -t, t)


def _make_sc_call(n_pad, k, t_dim, c_dim, num_bases):
    row = t_dim * c_dim
    wrow = row // 2
    n_chunks = wrow // LANES
    n_ch = c_dim // 32
    n_blocks = n_pad // NB
    bpw = n_blocks // NW
    idx_w = NB * k
    bw_row = num_bases * idx_w

    mesh = plsc.VectorSubcoreMesh(core_axis_name="c", subcore_axis_name="s")

    @functools.partial(
        pl.kernel,
        out_type=jax.ShapeDtypeStruct((n_pad, row), jnp.float32),
        mesh=mesh,
        scratch_types=[
            pltpu.VMEM((bpw, idx_w), jnp.int32),
            pltpu.VMEM((bpw, idx_w), jnp.float32),
            pltpu.VMEM((bpw, bw_row), jnp.float32),
            pltpu.VMEM((num_bases * c_dim // 128, 128), jnp.float32),
            pltpu.VMEM((idx_w, wrow), jnp.int32),
            pltpu.VMEM((idx_w, wrow), jnp.int32),
            pltpu.VMEM((NB, row), jnp.float32),
            pltpu.VMEM((NB, row), jnp.float32),
            pltpu.SemaphoreType.DMA,
            pltpu.SemaphoreType.DMA,
            pltpu.SemaphoreType.DMA,
            pltpu.SemaphoreType.DMA,
        ],
    )
    def sc_call(z_hbm, idx_hbm, adj_hbm, bw_hbm, cc_hbm, out_hbm,
                idxw, adjw, bww, ccw, rows0, rows1, ob0, ob1,
                gsem0, gsem1, osem0, osem1):
        wid = lax.axis_index("s") * NC + lax.axis_index("c")
        blk0 = wid * bpw
        node0 = blk0 * NB

        pltpu.sync_copy(idx_hbm.at[pl.ds(blk0, bpw)], idxw)
        pltpu.sync_copy(adj_hbm.at[pl.ds(blk0, bpw)], adjw)
        pltpu.sync_copy(bw_hbm.at[pl.ds(blk0, bpw)], bww)
        pltpu.sync_copy(cc_hbm, ccw)

        ccv = []
        for b_ in range(num_bases):
            ccv.append([])
            for ch in range(n_ch):
                p0 = b_ * c_dim + ch * 32
                ccv[b_].append(
                    (ccw[p0 // 128, pl.ds(p0 % 128, LANES)],
                     ccw[(p0 + LANES) // 128, pl.ds((p0 + LANES) % 128, LANES)])
                )

        mask_hi = jnp.int32(-65536)

        rowbufs = (rows0, rows1)
        obufs = (ob0, ob1)
        gsems = (gsem0, gsem1)
        osems = (osem0, osem1)

        pltpu.async_copy(z_hbm.at[idxw.at[0]], rows0, gsem0)
        pltpu.async_copy(z_hbm.at[idxw.at[1]], rows1, gsem1)

        def process(cur, b):
            rows = rowbufs[b]
            ob = obufs[b]
            pltpu.make_async_copy(z_hbm.at[idxw.at[cur]], rows, gsems[b]).wait()

            @pl.when(cur >= 2)
            def _():
                pltpu.make_async_copy(
                    ob, out_hbm.at[pl.ds(node0 + (cur - 2) * NB, NB)], osems[b]
                ).wait()

            def ibody(i, carry):
                adjv = adjw[cur, pl.ds(i * k, k)]
                sv = [
                    adjv * bww[cur, pl.ds(b_ * idx_w + i * k, k)]
                    for b_ in range(num_bases)
                ]
                zero = jnp.zeros((LANES,), jnp.float32)
                accs = [[zero, zero] for _ in range(n_chunks)]
                for j in range(k):
                    s = [sv[b_][j] for b_ in range(num_bases)]
                    wv = []
                    for ch in range(n_ch):
                        pair = []
                        for par in range(2):
                            w = s[0] * ccv[0][ch][par]
                            for b_ in range(1, num_bases):
                                w = w + s[b_] * ccv[b_][ch][par]
                            pair.append(w)
                        wv.append(pair)
                    r = i * k + j
                    for q in range(n_chunks):
                        v = rows[r, pl.ds(q * LANES, LANES)]
                        ze = lax.bitcast_convert_type(v << 16, jnp.float32)
                        zo = lax.bitcast_convert_type(v & mask_hi, jnp.float32)
                        ch = q % n_ch
                        accs[q][0] = accs[q][0] + ze * wv[ch][0]
                        accs[q][1] = accs[q][1] + zo * wv[ch][1]
                for q in range(n_chunks):
                    ob[i, pl.ds(q * 32, LANES)] = _tanh16(accs[q][0])
                    ob[i, pl.ds(q * 32 + LANES, LANES)] = _tanh16(accs[q][1])
                return carry

            lax.fori_loop(0, NB, ibody, 0)

            pltpu.async_copy(ob, out_hbm.at[pl.ds(node0 + cur * NB, NB)], osems[b])

            @pl.when(cur + 2 < bpw)
            def _():
                pltpu.async_copy(z_hbm.at[idxw.at[cur + 2]], rowbufs[b], gsems[b])

        def outer(g, carry):
            for b in range(2):
                process(g * 2 + b, b)
            return carry

        lax.fori_loop(0, bpw // 2, outer, 0)

        pltpu.make_async_copy(
            ob0, out_hbm.at[pl.ds(node0 + (bpw - 2) * NB, NB)], osem0
        ).wait()
        pltpu.make_async_copy(
            ob1, out_hbm.at[pl.ds(node0 + (bpw - 1) * NB, NB)], osem1
        ).wait()

    return sc_call


def kernel(z, neighbor_indices, adjacency, basis_weights, channel_coeffs):
    b_dim, n, t_dim, c_dim = z.shape
    k = neighbor_indices.shape[1]
    num_bases = channel_coeffs.shape[1]
    row = t_dim * c_dim

    n_pad = ((n + NW * NB - 1) // (NW * NB)) * (NW * NB)
    pad = n_pad - n
    n_blocks = n_pad // NB

    zb = z.reshape(n, row // 2, 2).astype(jnp.bfloat16)
    zw = lax.bitcast_convert_type(zb, jnp.uint32).astype(jnp.int32)

    idx = jnp.pad(neighbor_indices.astype(jnp.int32), ((0, pad), (0, 0)))
    idx = idx.reshape(n_blocks, NB * k)
    adj = jnp.pad(adjacency[:, :k], ((0, pad), (0, 0))).reshape(n_blocks, NB * k)
    bw = jnp.pad(basis_weights[:, :, :k], ((0, 0), (0, pad), (0, 0)))
    bw = jnp.transpose(bw.reshape(num_bases, n_blocks, NB * k), (1, 0, 2))
    bw = bw.reshape(n_blocks, num_bases * NB * k)
    cc = jnp.transpose(channel_coeffs).reshape(num_bases, c_dim // 32, LANES, 2)
    cc = jnp.transpose(cc, (0, 1, 3, 2)).reshape(num_bases * c_dim // 128, 128)

    sc_call = _make_sc_call(n_pad, k, t_dim, c_dim, num_bases)
    out = sc_call(zw, idx, adj, bw, cc)
    out = out[:n].reshape(n, row // 32, 2, LANES)
    out = jnp.transpose(out, (0, 1, 3, 2)).reshape(n, row)
    return out.reshape(b_dim, n, t_dim, c_dim)


# speedup vs baseline: 2.8110x; 1.0902x over previous
import functools

import jax
import jax.numpy as jnp
from jax import lax
from jax.experimental import pallas as pl
from jax.experimental.pallas import tpu as pltpu
from jax.experimental.pallas import tpu_sc as plsc

NC = 2
NS = 16
NW = NC * NS
LANES = 16
NB = 8
BPW0 = 24
BPW1 = 56


def _tanh16(x):
    ax = jnp.abs(x)
    e = jnp.exp(-2.0 * ax)
    t = (1.0 - e) / (1.0 + e)
    return jnp.where(x < 0.0, -t, t)


def _make_sc_call(n_pad, k, t_dim, c_dim, num_bases):
    row = t_dim * c_dim
    n_slots = row // LANES
    n_cgroups = c_dim // LANES
    n_blocks = n_pad // NB
    idx_w = NB * k
    bpw_max = max(BPW0, BPW1)
    n_blocks_stage = n_blocks + bpw_max - min(BPW0, BPW1)

    mesh = plsc.VectorSubcoreMesh(core_axis_name="c", subcore_axis_name="s")

    @functools.partial(
        pl.kernel,
        out_type=jax.ShapeDtypeStruct((n_pad, row), jnp.float32),
        mesh=mesh,
        scratch_types=[
            pltpu.VMEM((bpw_max, idx_w), jnp.int32),
            pltpu.VMEM((bpw_max, idx_w), jnp.float32),
            pltpu.VMEM((num_bases, bpw_max, idx_w), jnp.float32),
            pltpu.VMEM((num_bases * c_dim // 128, 128), jnp.float32),
            pltpu.VMEM((idx_w, row), jnp.float32),
            pltpu.VMEM((idx_w, row), jnp.float32),
            pltpu.VMEM((NB, row), jnp.float32),
            pltpu.VMEM((NB, row), jnp.float32),
            pltpu.SemaphoreType.DMA,
            pltpu.SemaphoreType.DMA,
            pltpu.SemaphoreType.DMA,
            pltpu.SemaphoreType.DMA,
        ],
    )
    def sc_call(z_hbm, idx_hbm, adj_hbm, bw_hbm, cc_hbm, out_hbm,
                idxw, adjw, bww, ccw, rows0, rows1, ob0, ob1,
                gsem0, gsem1, osem0, osem1):
        c = lax.axis_index("c")
        s = lax.axis_index("s")
        on0 = c == 0
        blk0 = jnp.where(on0, s * BPW0, NS * BPW0 + s * BPW1)
        mybpw = jnp.where(on0, BPW0, BPW1)
        node0 = blk0 * NB

        pltpu.sync_copy(idx_hbm.at[pl.ds(blk0, bpw_max)], idxw)
        pltpu.sync_copy(adj_hbm.at[pl.ds(blk0, bpw_max)], adjw)
        for b_ in range(num_bases):
            pltpu.sync_copy(bw_hbm.at[b_, pl.ds(blk0, bpw_max)], bww.at[b_])
        pltpu.sync_copy(cc_hbm, ccw)

        ccv = []
        for b_ in range(num_bases):
            ccv.append([])
            for g in range(n_cgroups):
                p = b_ * c_dim + g * LANES
                ccv[b_].append(ccw[p // 128, pl.ds(p % 128, LANES)])

        rowbufs = (rows0, rows1)
        obufs = (ob0, ob1)
        gsems = (gsem0, gsem1)
        osems = (osem0, osem1)

        pltpu.async_copy(z_hbm.at[idxw.at[0]], rows0, gsem0)
        pltpu.async_copy(z_hbm.at[idxw.at[1]], rows1, gsem1)

        def process(cur, b):
            rows = rowbufs[b]
            ob = obufs[b]
            pltpu.make_async_copy(z_hbm.at[idxw.at[cur]], rows, gsems[b]).wait()

            @pl.when(cur >= 2)
            def _():
                pltpu.make_async_copy(
                    ob, out_hbm.at[pl.ds(node0 + (cur - 2) * NB, NB)], osems[b]
                ).wait()

            def ibody(i, carry):
                adjv = adjw[cur, pl.ds(i * k, k)]
                sv = [
                    adjv * bww[b_, cur, pl.ds(i * k, k)]
                    for b_ in range(num_bases)
                ]
                zero = jnp.zeros((LANES,), jnp.float32)
                accs = [zero] * n_slots
                for j in range(k):
                    s_ = [sv[b_][j] for b_ in range(num_bases)]
                    wv = []
                    for g in range(n_cgroups):
                        w = s_[0] * ccv[0][g]
                        for b_ in range(1, num_bases):
                            w = w + s_[b_] * ccv[b_][g]
                        wv.append(w)
                    r = i * k + j
                    for sl in range(n_slots):
                        accs[sl] = (
                            accs[sl]
                            + rows[r, pl.ds(sl * LANES, LANES)] * wv[sl % n_cgroups]
                        )
                for sl in range(n_slots):
                    ob[i, pl.ds(sl * LANES, LANES)] = _tanh16(accs[sl])
                return carry

            lax.fori_loop(0, NB, ibody, 0)

            pltpu.async_copy(ob, out_hbm.at[pl.ds(node0 + cur * NB, NB)], osems[b])

            @pl.when(cur + 2 < mybpw)
            def _():
                pltpu.async_copy(z_hbm.at[idxw.at[cur + 2]], rowbufs[b], gsems[b])

        def outer(g, carry):
            for b in range(2):
                process(g * 2 + b, b)
            return carry

        lax.fori_loop(0, mybpw // 2, outer, 0)

        pltpu.make_async_copy(
            ob0, out_hbm.at[pl.ds(node0 + (mybpw - 2) * NB, NB)], osem0
        ).wait()
        pltpu.make_async_copy(
            ob1, out_hbm.at[pl.ds(node0 + (mybpw - 1) * NB, NB)], osem1
        ).wait()

    return sc_call


def kernel(z, neighbor_indices, adjacency, basis_weights, channel_coeffs):
    b_dim, n, t_dim, c_dim = z.shape
    k = neighbor_indices.shape[1]
    num_bases = channel_coeffs.shape[1]
    row = t_dim * c_dim

    per_pair = BPW0 + BPW1
    n_pad = ((n + NS * per_pair * NB - 1) // (NS * per_pair * NB)) * (NS * per_pair * NB)
    n_blocks = n_pad // NB
    n_blocks_stage = n_blocks + max(BPW0, BPW1) - min(BPW0, BPW1)
    pad = n_blocks_stage * NB - n

    z_flat = z.reshape(n, row)
    idx = jnp.pad(neighbor_indices.astype(jnp.int32), ((0, pad), (0, 0)))
    idx = idx.reshape(n_blocks_stage, NB * k)
    adj = jnp.pad(adjacency[:, :k], ((0, pad), (0, 0)))
    adj = adj.reshape(n_blocks_stage, NB * k)
    bw = jnp.pad(basis_weights[:, :, :k], ((0, 0), (0, pad), (0, 0)))
    bw = bw.reshape(num_bases, n_blocks_stage, NB * k)
    cc = jnp.transpose(channel_coeffs).reshape(num_bases * c_dim // 128, 128)

    sc_call = _make_sc_call(n_pad, k, t_dim, c_dim, num_bases)
    out = sc_call(z_flat, idx, adj, bw, cc)
    return out[:n].reshape(b_dim, n, t_dim, c_dim)
